# baseline (speedup 1.0000x reference)
import functools

import jax
import jax.numpy as jnp
import numpy as np
from jax import lax
from jax.experimental import pallas as pl
from jax.experimental.pallas import tpu as pltpu
from jax.experimental.pallas import tpu_sc as plsc

N = 10000
E = 320000
D = 128
DA = 8
DE = 16
FC_H = 8
LOG2 = float(np.log(2.0))
INV_SQRT_D = 1.0 / float(np.sqrt(float(D)))
INV_SQRT_DE = 1.0 / float(np.sqrt(float(DE)))
INV_SQRT_H = 1.0 / float(np.sqrt(float(FC_H)))
INV_SQRT_AVG = 1.0 / float(np.sqrt(32.0))
INV_SQRT_DDA = 1.0 / float(np.sqrt(float(D * DA)))

NC = 2
NS = 16
CHUNK = 80
EH = E // NC
NCH = EH // CHUNK
ROWS_A = 624
ROWS_LAST = N - (NS - 1) * ROWS_A
ZR = 16


def _ssp(x):
    return jax.nn.softplus(x) - LOG2


def _x_body(nf_ref, w1_ref, o_ref):
    o_ref[...] = jnp.dot(nf_ref[...], w1_ref[...],
                         preferred_element_type=jnp.float32) * INV_SQRT_D


def _node_linear(nf, w1):
    bn = 2000
    return pl.pallas_call(
        _x_body,
        grid=(N // bn,),
        in_specs=[
            pl.BlockSpec((bn, D), lambda i: (i, 0)),
            pl.BlockSpec((D, D), lambda i: (0, 0)),
        ],
        out_specs=pl.BlockSpec((bn, D), lambda i: (i, 0)),
        out_shape=jax.ShapeDtypeStruct((N, D), jnp.float32),
    )(nf, w1)


_BE = 3200
_GE = _BE // 128
_GP = _BE // 80


def _rne_bf16_bits(x):
    u = lax.bitcast_convert_type(x, jnp.uint32)
    return (u + jnp.uint32(0x7FFF) + ((u >> 16) & jnp.uint32(1))) >> 16


def _cw_body(eet_ref, ea2_ref, wr1t_ref, wr2_ref, o_ref):
    ee = jnp.transpose(eet_ref[...])
    h = _ssp(lax.dot_general(ee, wr1t_ref[...],
                             (((1,), (1,)), ((), ())),
                             preferred_element_type=jnp.float32) * INV_SQRT_DE)
    w = jnp.dot(h, wr2_ref[...], preferred_element_type=jnp.float32) * INV_SQRT_H
    ea_t = jnp.transpose(ea2_ref[0])
    ea_col = jnp.concatenate([ea_t[:, g:g + 1] for g in range(_GE)], axis=0)
    cw = w * ea_col
    for g in range(_GP):
        r0 = g * 80
        lo = _rne_bf16_bits(cw[r0:r0 + 40, :])
        hi = _rne_bf16_bits(cw[r0 + 40:r0 + 80, :])
        o_ref[g * 40:(g + 1) * 40, :] = lax.bitcast_convert_type(
            lo | (hi << 16), jnp.int32)


def _edge_coeffs(ee, ea, wr1, wr2):
    ee_t = jnp.transpose(ee)
    ea2 = jnp.reshape(ea, (E // _BE, _GE, 128))
    wr1_t = jnp.transpose(wr1)
    return pl.pallas_call(
        _cw_body,
        grid=(E // _BE,),
        in_specs=[
            pl.BlockSpec((DE, _BE), lambda i: (0, i)),
            pl.BlockSpec((1, _GE, 128), lambda i: (i, 0, 0)),
            pl.BlockSpec((FC_H, DE), lambda i: (0, 0)),
            pl.BlockSpec((FC_H, D), lambda i: (0, 0)),
        ],
        out_specs=pl.BlockSpec((_BE // 2, D), lambda i: (i, 0)),
        out_shape=jax.ShapeDtypeStruct((E // 2, D), jnp.int32),
    )(ee_t, ea2, wr1_t, wr2)


def _sc_edge_body(x_hbm, cw_hbm, src_hbm, dst_hbm, out_hbm, agg_sh,
                  s0, s1, d0, d1, c0, c1, g0, g1,
                  ss0, ss1, sd0, sd1, sc0, sc1, sg0, sg1):
    cid = lax.axis_index("c")
    sid = lax.axis_index("s")
    srcs, dsts, pks, xgs = [s0, s1], [d0, d1], [c0, c1], [g0, g1]
    sem_s, sem_d, sem_c, sem_g = [ss0, ss1], [sd0, sd1], [sc0, sc1], [sg0, sg1]

    def zrow(r, _):
        for c in range(D // 16):
            g0[r, pl.ds(c * 16, 16)] = jnp.zeros((16,), jnp.float32)
        return 0
    lax.fori_loop(0, ZR, zrow, 0)
    row0 = pl.multiple_of(sid * ROWS_A, 8)
    nz = jnp.where(sid == NS - 1, ROWS_LAST // ZR, ROWS_A // ZR)

    def zcopy(k, _):
        pltpu.sync_copy(g0.at[pl.ds(0, ZR)],
                        agg_sh.at[pl.ds(pl.multiple_of(row0 + k * ZR, 8), ZR)])
        return 0
    lax.fori_loop(0, nz, zcopy, 0)
    plsc.subcore_barrier()

    q, r = NCH // NS, NCH % NS
    nj = jnp.where(sid < r, q + 1, q)

    def ebase(j):
        return cid * EH + (sid + j * NS) * CHUNK

    def prefetch(j, b):
        base = ebase(j)
        pltpu.async_copy(src_hbm.at[pl.ds(base, CHUNK)], srcs[b], sem_s[b])
        pltpu.async_copy(dst_hbm.at[pl.ds(base, CHUNK)], dsts[b], sem_d[b])
        pltpu.async_copy(cw_hbm.at[pl.ds(pl.multiple_of(base // 2, 8), CHUNK // 2)],
                         pks[b], sem_c[b])

    prefetch(0, 0)
    pltpu.make_async_copy(src_hbm.at[pl.ds(ebase(0), CHUNK)], srcs[0], sem_s[0]).wait()
    pltpu.async_copy(x_hbm.at[srcs[0]], xgs[0], sem_g[0])

    def pair(t, _):
        for b in (0, 1):
            j = 2 * t + b
            nb = 1 - b

            @pl.when(j < nj)
            def _process():
                @pl.when(j + 1 < nj)
                def _pf():
                    prefetch(j + 1, nb)

                pltpu.make_async_copy(dst_hbm.at[pl.ds(ebase(j), CHUNK)],
                                      dsts[b], sem_d[b]).wait()
                pltpu.make_async_copy(
                    cw_hbm.at[pl.ds(pl.multiple_of(ebase(j) // 2, 8), CHUNK // 2)],
                    pks[b], sem_c[b]).wait()
                pltpu.make_async_copy(x_hbm.at[srcs[b]], xgs[b], sem_g[b]).wait()

                def mulrow(q, _):
                    for c in range(D // 16):
                        sl = pl.ds(c * 16, 16)
                        pkv = pks[b][q, sl]
                        wlo = lax.bitcast_convert_type(pkv << 16, jnp.float32)
                        whi = lax.bitcast_convert_type(
                            pkv & jnp.int32(-65536), jnp.float32)
                        xgs[b][q, sl] = xgs[b][q, sl] * wlo
                        xgs[b][q + CHUNK // 2, sl] = (
                            xgs[b][q + CHUNK // 2, sl] * whi)
                    return 0
                lax.fori_loop(0, CHUNK // 2, mulrow, 0)

                @pl.when(j + 1 < nj)
                def _gather_next():
                    pltpu.make_async_copy(src_hbm.at[pl.ds(ebase(j + 1), CHUNK)],
                                          srcs[nb], sem_s[nb]).wait()
                    pltpu.async_copy(x_hbm.at[srcs[nb]], xgs[nb], sem_g[nb])

                pltpu.sync_copy(xgs[b], agg_sh.at[dsts[b]], add=True)
        return 0
    lax.fori_loop(0, (q + 2) // 2, pair, 0)
    plsc.subcore_barrier()

    out0 = pl.multiple_of(cid * N + row0, 8)

    @pl.when(sid < NS - 1)
    def _dump_a():
        pltpu.sync_copy(agg_sh.at[pl.ds(row0, ROWS_A)],
                        out_hbm.at[pl.ds(out0, ROWS_A)])

    @pl.when(sid == NS - 1)
    def _dump_last():
        pltpu.sync_copy(agg_sh.at[pl.ds(row0, ROWS_LAST)],
                        out_hbm.at[pl.ds(out0, ROWS_LAST)])


def _sc_edge_aggregate(x, cw, src, dst):
    mesh = plsc.VectorSubcoreMesh(core_axis_name="c", subcore_axis_name="s")
    return pl.kernel(
        _sc_edge_body,
        mesh=mesh,
        out_type=jax.ShapeDtypeStruct((NC * N, D), jnp.float32),
        scratch_types=(
            [pltpu.VMEM_SHARED((N, D), jnp.float32)]
            + [pltpu.VMEM((CHUNK,), jnp.int32) for _ in range(4)]
            + [pltpu.VMEM((CHUNK // 2, D), jnp.int32) for _ in range(2)]
            + [pltpu.VMEM((CHUNK, D), jnp.float32) for _ in range(2)]
            + [pltpu.SemaphoreType.DMA for _ in range(8)]
        ),
    )(x, cw, src, dst)


def _post_body(pa_ref, pb_ref, nf_ref, na_ref, w2_ref, wsc_ref, o_ref):
    agg = (pa_ref[...] + pb_ref[...]) * INV_SQRT_AVG
    t = jnp.dot(agg, w2_ref[...], preferred_element_type=jnp.float32) * INV_SQRT_D
    nf = nf_ref[...]
    na = na_ref[...]
    acc = jnp.zeros_like(t)
    for v in range(DA):
        wv = wsc_ref[:, v, :]
        yv = jnp.dot(nf, wv, preferred_element_type=jnp.float32)
        acc = acc + na[:, v:v + 1] * yv
    o_ref[...] = nf + _ssp(t + acc * INV_SQRT_DDA)


def _postprocess(partials, nf, na, w2, wsc):
    bn = 2000
    nb = N // bn
    return pl.pallas_call(
        _post_body,
        grid=(nb,),
        in_specs=[
            pl.BlockSpec((bn, D), lambda i: (i, 0)),
            pl.BlockSpec((bn, D), lambda i, _nb=nb: (i + _nb, 0)),
            pl.BlockSpec((bn, D), lambda i: (i, 0)),
            pl.BlockSpec((bn, DA), lambda i: (i, 0)),
            pl.BlockSpec((D, D), lambda i: (0, 0)),
            pl.BlockSpec((D, DA, D), lambda i: (0, 0, 0)),
        ],
        out_specs=pl.BlockSpec((bn, D), lambda i: (i, 0)),
        out_shape=jax.ShapeDtypeStruct((N, D), jnp.float32),
    )(partials, partials, nf, na, w2, wsc)


def kernel(node_features, node_attrs, edge_embedding, edge_attrs, edge_index,
           W1, Wr1, Wr2, W2, Wsc):
    src = edge_index[0]
    dst = edge_index[1]
    x = _node_linear(node_features, W1)
    cw = _edge_coeffs(edge_embedding, edge_attrs, Wr1, Wr2)
    partials = _sc_edge_aggregate(x, cw, src, dst)
    return _postprocess(partials, node_features, node_attrs, W2, Wsc)


# speedup vs baseline: 1.1667x; 1.1667x over previous
import functools

import jax
import jax.numpy as jnp
import numpy as np
from jax import lax
from jax.experimental import pallas as pl
from jax.experimental.pallas import tpu as pltpu
from jax.experimental.pallas import tpu_sc as plsc

N = 10000
E = 320000
D = 128
DA = 8
DE = 16
FC_H = 8
LOG2 = float(np.log(2.0))
INV_SQRT_D = 1.0 / float(np.sqrt(float(D)))
INV_SQRT_DE = 1.0 / float(np.sqrt(float(DE)))
INV_SQRT_H = 1.0 / float(np.sqrt(float(FC_H)))
INV_SQRT_AVG = 1.0 / float(np.sqrt(32.0))
INV_SQRT_DDA = 1.0 / float(np.sqrt(float(D * DA)))

NC = 2
NS = 16
CHUNK = 80
EH = E // NC
NCH = EH // CHUNK
ROWS_A = 624
ROWS_LAST = N - (NS - 1) * ROWS_A
ZR = 16


def _ssp(x):
    return jax.nn.softplus(x) - LOG2


def _x_body(nf_ref, w1_ref, o_ref):
    o_ref[...] = jnp.dot(nf_ref[...], w1_ref[...],
                         preferred_element_type=jnp.float32) * INV_SQRT_D


def _node_linear(nf, w1):
    bn = 2000
    return pl.pallas_call(
        _x_body,
        grid=(N // bn,),
        in_specs=[
            pl.BlockSpec((bn, D), lambda i: (i, 0)),
            pl.BlockSpec((D, D), lambda i: (0, 0)),
        ],
        out_specs=pl.BlockSpec((bn, D), lambda i: (i, 0)),
        out_shape=jax.ShapeDtypeStruct((N, D), jnp.float32),
    )(nf, w1)


_BE = 3200
_GE = _BE // 128
_GP = _BE // 80


def _cw_body(eet_ref, ea2_ref, wr1t_ref, wr2_ref, o_ref):
    ee = jnp.transpose(eet_ref[...])
    h = _ssp(lax.dot_general(ee, wr1t_ref[...],
                             (((1,), (1,)), ((), ())),
                             preferred_element_type=jnp.float32) * INV_SQRT_DE)
    w = jnp.dot(h, wr2_ref[...], preferred_element_type=jnp.float32) * INV_SQRT_H
    ea_t = jnp.transpose(ea2_ref[0])
    ea_col = jnp.concatenate([ea_t[:, g:g + 1] for g in range(_GE)], axis=0)
    cw = w * ea_col
    cwv = jnp.reshape(cw, (_GP, 80, D))
    u_lo = lax.bitcast_convert_type(cwv[:, :40, :], jnp.uint32)
    u_hi = lax.bitcast_convert_type(cwv[:, 40:, :], jnp.uint32)
    pk = (((u_lo + jnp.uint32(0x8000)) >> 16)
          | ((u_hi + jnp.uint32(0x8000)) & jnp.uint32(0xFFFF0000)))
    o_ref[...] = lax.bitcast_convert_type(
        jnp.reshape(pk, (_BE // 2, D)), jnp.int32)


def _edge_coeffs(ee, ea, wr1, wr2):
    ee_t = jnp.transpose(ee)
    ea2 = jnp.reshape(ea, (E // _BE, _GE, 128))
    wr1_t = jnp.transpose(wr1)
    return pl.pallas_call(
        _cw_body,
        grid=(E // _BE,),
        in_specs=[
            pl.BlockSpec((DE, _BE), lambda i: (0, i)),
            pl.BlockSpec((1, _GE, 128), lambda i: (i, 0, 0)),
            pl.BlockSpec((FC_H, DE), lambda i: (0, 0)),
            pl.BlockSpec((FC_H, D), lambda i: (0, 0)),
        ],
        out_specs=pl.BlockSpec((_BE // 2, D), lambda i: (i, 0)),
        out_shape=jax.ShapeDtypeStruct((E // 2, D), jnp.int32),
    )(ee_t, ea2, wr1_t, wr2)


def _sc_edge_body(x_hbm, cw_hbm, src_hbm, dst_hbm, out_hbm, agg_sh,
                  s0, s1, s2, d0, d1, d2, c0, c1, c2, g0, g1, g2,
                  ss0, ss1, ss2, sd0, sd1, sd2, sc0, sc1, sc2,
                  sg0, sg1, sg2, sa0, sa1, sa2):
    cid = lax.axis_index("c")
    sid = lax.axis_index("s")
    srcs, dsts, pks, xgs = [s0, s1, s2], [d0, d1, d2], [c0, c1, c2], [g0, g1, g2]
    sem_s, sem_d, sem_c = [ss0, ss1, ss2], [sd0, sd1, sd2], [sc0, sc1, sc2]
    sem_g, sem_a = [sg0, sg1, sg2], [sa0, sa1, sa2]

    def zrow(r, _):
        for c in range(D // 16):
            g0[r, pl.ds(c * 16, 16)] = jnp.zeros((16,), jnp.float32)
        return 0
    lax.fori_loop(0, ZR, zrow, 0)
    row0 = pl.multiple_of(sid * ROWS_A, 8)
    nz = jnp.where(sid == NS - 1, ROWS_LAST // ZR, ROWS_A // ZR)

    def zcopy(k, _):
        pltpu.sync_copy(g0.at[pl.ds(0, ZR)],
                        agg_sh.at[pl.ds(pl.multiple_of(row0 + k * ZR, 8), ZR)])
        return 0
    lax.fori_loop(0, nz, zcopy, 0)
    plsc.subcore_barrier()

    NJ = NCH // NS
    assert NCH % NS == 0

    def ebase(j):
        return cid * EH + (sid + j * NS) * CHUNK

    def prefetch(j, b):
        base = ebase(j)
        pltpu.async_copy(src_hbm.at[pl.ds(base, CHUNK)], srcs[b], sem_s[b])
        pltpu.async_copy(dst_hbm.at[pl.ds(base, CHUNK)], dsts[b], sem_d[b])
        pltpu.async_copy(cw_hbm.at[pl.ds(pl.multiple_of(base // 2, 8), CHUNK // 2)],
                         pks[b], sem_c[b])

    def gather(j, b):
        pltpu.make_async_copy(src_hbm.at[pl.ds(ebase(j), CHUNK)],
                              srcs[b], sem_s[b]).wait()
        pltpu.async_copy(x_hbm.at[srcs[b]], xgs[b], sem_g[b])

    for jj in (0, 1):
        prefetch(jj, jj)
        gather(jj, jj)

    def triple(t, _):
        for b in (0, 1, 2):
            j = 3 * t + b
            s2 = (b + 2) % 3

            @pl.when(j < NJ)
            def _process():
                @pl.when(j >= 1)
                def _drain_prev_scatter():
                    pltpu.make_async_copy(xgs[s2], agg_sh.at[dsts[s2]],
                                          sem_a[s2]).wait()

                @pl.when(j + 2 < NJ)
                def _pf():
                    prefetch(j + 2, s2)

                pltpu.make_async_copy(dst_hbm.at[pl.ds(ebase(j), CHUNK)],
                                      dsts[b], sem_d[b]).wait()
                pltpu.make_async_copy(
                    cw_hbm.at[pl.ds(pl.multiple_of(ebase(j) // 2, 8), CHUNK // 2)],
                    pks[b], sem_c[b]).wait()
                pltpu.make_async_copy(x_hbm.at[srcs[b]], xgs[b], sem_g[b]).wait()

                def mulrow(q, _):
                    for c in range(D // 16):
                        sl = pl.ds(c * 16, 16)
                        pkv = pks[b][q, sl]
                        wlo = lax.bitcast_convert_type(pkv << 16, jnp.float32)
                        whi = lax.bitcast_convert_type(
                            pkv & jnp.int32(-65536), jnp.float32)
                        xgs[b][q, sl] = xgs[b][q, sl] * wlo
                        xgs[b][q + CHUNK // 2, sl] = (
                            xgs[b][q + CHUNK // 2, sl] * whi)
                    return 0
                lax.fori_loop(0, CHUNK // 2, mulrow, 0)

                @pl.when(j + 2 < NJ)
                def _gather_next():
                    gather(j + 2, s2)

                pltpu.async_copy(xgs[b], agg_sh.at[dsts[b]], sem_a[b], add=True)
        return 0
    lax.fori_loop(0, (NJ + 2) // 3, triple, 0)
    lb = (NJ - 1) % 3
    pltpu.make_async_copy(xgs[lb], agg_sh.at[dsts[lb]], sem_a[lb]).wait()
    plsc.subcore_barrier()

    out0 = pl.multiple_of(cid * N + row0, 8)

    @pl.when(sid < NS - 1)
    def _dump_a():
        pltpu.sync_copy(agg_sh.at[pl.ds(row0, ROWS_A)],
                        out_hbm.at[pl.ds(out0, ROWS_A)])

    @pl.when(sid == NS - 1)
    def _dump_last():
        pltpu.sync_copy(agg_sh.at[pl.ds(row0, ROWS_LAST)],
                        out_hbm.at[pl.ds(out0, ROWS_LAST)])


def _sc_edge_aggregate(x, cw, src, dst):
    mesh = plsc.VectorSubcoreMesh(core_axis_name="c", subcore_axis_name="s")
    return pl.kernel(
        _sc_edge_body,
        mesh=mesh,
        out_type=jax.ShapeDtypeStruct((NC * N, D), jnp.float32),
        scratch_types=(
            [pltpu.VMEM_SHARED((N, D), jnp.float32)]
            + [pltpu.VMEM((CHUNK,), jnp.int32) for _ in range(6)]
            + [pltpu.VMEM((CHUNK // 2, D), jnp.int32) for _ in range(3)]
            + [pltpu.VMEM((CHUNK, D), jnp.float32) for _ in range(3)]
            + [pltpu.SemaphoreType.DMA for _ in range(15)]
        ),
    )(x, cw, src, dst)


def _post_body(pa_ref, pb_ref, nf_ref, na_ref, w2_ref, wsc_ref, o_ref):
    agg = (pa_ref[...] + pb_ref[...]) * INV_SQRT_AVG
    t = jnp.dot(agg, w2_ref[...], preferred_element_type=jnp.float32) * INV_SQRT_D
    nf = nf_ref[...]
    na = na_ref[...]
    acc = jnp.zeros_like(t)
    for v in range(DA):
        wv = wsc_ref[:, v, :]
        yv = jnp.dot(nf, wv, preferred_element_type=jnp.float32)
        acc = acc + na[:, v:v + 1] * yv
    o_ref[...] = nf + _ssp(t + acc * INV_SQRT_DDA)


def _postprocess(partials, nf, na, w2, wsc):
    bn = 2000
    nb = N // bn
    return pl.pallas_call(
        _post_body,
        grid=(nb,),
        in_specs=[
            pl.BlockSpec((bn, D), lambda i: (i, 0)),
            pl.BlockSpec((bn, D), lambda i, _nb=nb: (i + _nb, 0)),
            pl.BlockSpec((bn, D), lambda i: (i, 0)),
            pl.BlockSpec((bn, DA), lambda i: (i, 0)),
            pl.BlockSpec((D, D), lambda i: (0, 0)),
            pl.BlockSpec((D, DA, D), lambda i: (0, 0, 0)),
        ],
        out_specs=pl.BlockSpec((bn, D), lambda i: (i, 0)),
        out_shape=jax.ShapeDtypeStruct((N, D), jnp.float32),
    )(partials, partials, nf, na, w2, wsc)


def kernel(node_features, node_attrs, edge_embedding, edge_attrs, edge_index,
           W1, Wr1, Wr2, W2, Wsc):
    src = edge_index[0]
    dst = edge_index[1]
    x = _node_linear(node_features, W1)
    cw = _edge_coeffs(edge_embedding, edge_attrs, Wr1, Wr2)
    partials = _sc_edge_aggregate(x, cw, src, dst)
    return _postprocess(partials, node_features, node_attrs, W2, Wsc)
